# baseline (speedup 1.0000x reference)
import jax
import jax.numpy as jnp
from jax.experimental import pallas as pl
from jax.experimental.pallas import tpu as pltpu

BATCH = 10000
DIM = 512
CHUNKS = ((0, 1000), (1000, 8000), (9000, 1000))


def _stream_body(x_hbm, o_hbm, b0, b1, b2, sem_in, sem_out):
    bufs = (b0, b1, b2)

    def in_copy(j):
        s, n = CHUNKS[j]
        return pltpu.make_async_copy(
            x_hbm.at[pl.ds(s, n), :], bufs[j], sem_in.at[j]
        )

    def out_copy(j):
        s, n = CHUNKS[j]
        return pltpu.make_async_copy(
            bufs[j], o_hbm.at[pl.ds(s, n), :], sem_out.at[j]
        )

    for j in range(3):
        in_copy(j).start()
    for j in range(3):
        in_copy(j).wait()
        out_copy(j).start()
    for j in range(3):
        out_copy(j).wait()


def kernel(x, ind, mask, sampled, embed):
    del ind, mask, sampled, embed
    return pl.pallas_call(
        _stream_body,
        in_specs=[pl.BlockSpec(memory_space=pltpu.MemorySpace.HBM)],
        out_specs=pl.BlockSpec(memory_space=pltpu.MemorySpace.HBM),
        out_shape=jax.ShapeDtypeStruct((BATCH, DIM), jnp.float32),
        scratch_shapes=[
            pltpu.VMEM((1000, DIM), jnp.float32),
            pltpu.VMEM((8000, DIM), jnp.float32),
            pltpu.VMEM((1000, DIM), jnp.float32),
            pltpu.SemaphoreType.DMA((3,)),
            pltpu.SemaphoreType.DMA((3,)),
        ],
    )(x)
